# baseline (speedup 1.0000x reference)
import functools

import jax
import jax.numpy as jnp
from jax import lax
from jax.experimental import pallas as pl
from jax.experimental.pallas import tpu as pltpu
from jax.experimental.pallas import tpu_sc as plsc

_THR = 0.05
_TEMP = 2.0
_KL_WEIGHT = 0.2

_NS = 8192
_NT = 8192
_C = 22

_SBLK = 1024
_TCHUNK = 1024
_RBLK = 1024


def _nn_body(st_ref, t_ref, oi_ref, od_ref):
    sx = st_ref[0:1, :]
    sy = st_ref[1:2, :]
    sz = st_ref[2:3, :]
    run_min = jnp.full((1, _SBLK), jnp.inf, jnp.float32)
    run_idx = jnp.zeros((1, _SBLK), jnp.int32)
    for c in range(_NT // _TCHUNK):
        rows = pl.ds(c * _TCHUNK, _TCHUNK)
        tx = t_ref[rows, 0:1]
        ty = t_ref[rows, 1:2]
        tz = t_ref[rows, 2:3]
        dx = tx - sx
        dy = ty - sy
        dz = tz - sz
        d2 = dx * dx + dy * dy + dz * dz
        cmin = jnp.min(d2, axis=0, keepdims=True)
        iota = lax.broadcasted_iota(jnp.int32, (_TCHUNK, _SBLK), 0)
        cidx = jnp.min(jnp.where(d2 == cmin, iota, _NT), axis=0,
                       keepdims=True) + c * _TCHUNK
        better = cmin < run_min
        run_idx = jnp.where(better, cidx, run_idx)
        run_min = jnp.minimum(run_min, cmin)
    oi_ref[...] = run_idx
    od_ref[...] = run_min


def _nn_cols(s_coord_t, t_coord):
    return pl.pallas_call(
        _nn_body,
        grid=(_NS // _SBLK,),
        in_specs=[
            pl.BlockSpec((3, _SBLK), lambda i: (0, i)),
            pl.BlockSpec((_NT, 3), lambda i: (0, 0)),
        ],
        out_specs=[pl.BlockSpec((1, _SBLK), lambda i: (0, i)),
                   pl.BlockSpec((1, _SBLK), lambda i: (0, i))],
        out_shape=[jax.ShapeDtypeStruct((1, _NS), jnp.int32),
                   jax.ShapeDtypeStruct((1, _NS), jnp.float32)],
    )(s_coord_t, t_coord)


def _make_sc_gather():
    info = plsc.get_sparse_core_info()
    nw = info.num_cores * info.num_subcores
    b_per_w = _NS // nw
    mesh = plsc.VectorSubcoreMesh(core_axis_name="c", subcore_axis_name="s")

    @functools.partial(
        pl.kernel,
        out_type=jax.ShapeDtypeStruct((_NS, 128), jnp.float32),
        mesh=mesh,
        scratch_types=[pltpu.VMEM((b_per_w,), jnp.int32),
                       pltpu.VMEM((b_per_w, 128), jnp.float32),
                       pltpu.SemaphoreType.DMA],
    )
    def gather_kernel(tl_hbm, idx_hbm, gl_hbm, idx_v, rows_l, sem_l):
        wid = lax.axis_index("s") * info.num_cores + lax.axis_index("c")
        base = wid * b_per_w
        pltpu.sync_copy(idx_hbm.at[pl.ds(base, b_per_w)], idx_v)
        pltpu.async_copy(tl_hbm.at[idx_v], rows_l, sem_l).wait()
        pltpu.sync_copy(rows_l, gl_hbm.at[pl.ds(base, b_per_w)])

    return gather_kernel


def _kl_body(d2_ref, sl_ref, gl_ref, o_ref):
    kl_sum = jnp.zeros((1, 1), jnp.float32)
    n_sum = jnp.zeros((1, 1), jnp.float32)
    inv_t = 1.0 / _TEMP
    for c in range(_NS // _RBLK):
        rows = pl.ds(c * _RBLK, _RBLK)
        dist = jnp.sqrt(d2_ref[rows, :])
        maskf = (dist <= _THR).astype(jnp.float32)
        sl = sl_ref[rows, :] * inv_t
        tl = gl_ref[rows, 0:_C] * inv_t
        sm = jnp.max(sl, axis=1, keepdims=True)
        s_lse = jnp.log(jnp.sum(jnp.exp(sl - sm), axis=1, keepdims=True)) + sm
        tm = jnp.max(tl, axis=1, keepdims=True)
        te = jnp.exp(tl - tm)
        tsum = jnp.sum(te, axis=1, keepdims=True)
        t_lse = jnp.log(tsum) + tm
        t_prob = te / tsum
        kl_per = jnp.sum(t_prob * ((tl - t_lse) - (sl - s_lse)),
                         axis=1, keepdims=True)
        kl_sum = kl_sum + jnp.sum(kl_per * maskf, keepdims=True)
        n_sum = n_sum + jnp.sum(maskf, keepdims=True)
    loss = jnp.where(n_sum > 0.0, kl_sum / jnp.maximum(n_sum, 1.0), 0.0)
    o_ref[...] = loss * (_TEMP * _TEMP * _KL_WEIGHT)


def _kl_loss(d2col, s_logits, g_logits):
    return pl.pallas_call(
        _kl_body,
        in_specs=[
            pl.BlockSpec((_NS, 1), lambda: (0, 0)),
            pl.BlockSpec((_NS, _C), lambda: (0, 0)),
            pl.BlockSpec((_NS, 128), lambda: (0, 0)),
        ],
        out_specs=pl.BlockSpec((1, 1), lambda: (0, 0)),
        out_shape=jax.ShapeDtypeStruct((1, 1), jnp.float32),
    )(d2col, s_logits, g_logits)


def kernel(s_coord, t_coord, s_logits, t_logits):
    col2d, d2row = _nn_cols(s_coord.T, t_coord)
    col = col2d.reshape(_NS)
    tl_pad = jnp.pad(t_logits, ((0, 0), (0, 128 - _C)))
    g_logits = _make_sc_gather()(tl_pad, col)
    out = _kl_loss(d2row.reshape(_NS, 1), s_logits, g_logits)
    return out[0, 0]


# speedup vs baseline: 1.2124x; 1.2124x over previous
import functools

import jax
import jax.numpy as jnp
from jax import lax
from jax.experimental import pallas as pl
from jax.experimental.pallas import tpu as pltpu
from jax.experimental.pallas import tpu_sc as plsc

_THR = 0.05
_TEMP = 2.0
_KL_WEIGHT = 0.2

_NS = 8192
_NT = 8192
_C = 22

_SBLK = 1024
_TCHUNK = 1024
_RBLK = 1024


def _nn_body(s_ref, tt_ref, oi_ref):
    s3 = s_ref[...]
    sx = s_ref[:, 0:1]
    sy = s_ref[:, 1:2]
    sz = s_ref[:, 2:3]
    s2 = sx * sx + sy * sy + sz * sz
    run_min = jnp.full((_SBLK, 1), jnp.inf, jnp.float32)
    run_idx = jnp.zeros((_SBLK, 1), jnp.int32)
    for c in range(_NT // _TCHUNK):
        cols = pl.ds(c * _TCHUNK, _TCHUNK)
        tx = tt_ref[0:1, cols]
        ty = tt_ref[1:2, cols]
        tz = tt_ref[2:3, cols]
        t2 = tx * tx + ty * ty + tz * tz
        dot = lax.dot_general(s3, tt_ref[:, cols],
                              (((1,), (0,)), ((), ())),
                              preferred_element_type=jnp.float32)
        d2 = s2 - 2.0 * dot + t2
        cmin = jnp.min(d2, axis=1, keepdims=True)
        iota = lax.broadcasted_iota(jnp.int32, (_SBLK, _TCHUNK), 1)
        cidx = jnp.min(jnp.where(d2 == cmin, iota, _NT), axis=1,
                       keepdims=True) + c * _TCHUNK
        better = cmin < run_min
        run_idx = jnp.where(better, cidx, run_idx)
        run_min = jnp.minimum(run_min, cmin)
    oi_ref[...] = run_idx


def _nn_cols(s_coord, t_coord_t):
    return pl.pallas_call(
        _nn_body,
        grid=(_NS // _SBLK,),
        in_specs=[
            pl.BlockSpec((_SBLK, 3), lambda i: (i, 0)),
            pl.BlockSpec((3, _NT), lambda i: (0, 0)),
        ],
        out_specs=pl.BlockSpec((_SBLK, 1), lambda i: (i, 0)),
        out_shape=jax.ShapeDtypeStruct((_NS, 1), jnp.int32),
        compiler_params=pltpu.CompilerParams(
            dimension_semantics=("parallel",)),
    )(s_coord, t_coord_t)


def _make_sc_gather():
    info = plsc.get_sparse_core_info()
    nw = info.num_cores * info.num_subcores
    b_per_w = _NS // nw
    mesh = plsc.VectorSubcoreMesh(core_axis_name="c", subcore_axis_name="s")

    @functools.partial(
        pl.kernel,
        out_type=[jax.ShapeDtypeStruct((_NS, 128), jnp.float32),
                  jax.ShapeDtypeStruct((_NS, 128), jnp.float32)],
        mesh=mesh,
        scratch_types=[pltpu.VMEM((b_per_w,), jnp.int32),
                       pltpu.VMEM((b_per_w, 128), jnp.float32),
                       pltpu.VMEM((b_per_w, 128), jnp.float32),
                       pltpu.SemaphoreType.DMA,
                       pltpu.SemaphoreType.DMA],
    )
    def gather_kernel(tl_hbm, tc_hbm, idx_hbm, gl_hbm, gc_hbm,
                      idx_v, rows_l, rows_c, sem_l, sem_c):
        wid = lax.axis_index("s") * info.num_cores + lax.axis_index("c")
        base = wid * b_per_w
        pltpu.sync_copy(idx_hbm.at[pl.ds(base, b_per_w)], idx_v)
        cp_l = pltpu.async_copy(tl_hbm.at[idx_v], rows_l, sem_l)
        cp_c = pltpu.async_copy(tc_hbm.at[idx_v], rows_c, sem_c)
        cp_l.wait()
        cp_c.wait()
        pltpu.sync_copy(rows_l, gl_hbm.at[pl.ds(base, b_per_w)])
        pltpu.sync_copy(rows_c, gc_hbm.at[pl.ds(base, b_per_w)])

    return gather_kernel


def _kl_body(sc_ref, sl_ref, gl_ref, gc_ref, o_ref):
    kl_sum = jnp.zeros((1, 1), jnp.float32)
    n_sum = jnp.zeros((1, 1), jnp.float32)
    inv_t = 1.0 / _TEMP
    for c in range(_NS // _RBLK):
        rows = pl.ds(c * _RBLK, _RBLK)
        sc = sc_ref[rows, :]
        gc = gc_ref[rows, 0:3]
        diff = sc - gc
        dist = jnp.sqrt(jnp.sum(diff * diff, axis=1, keepdims=True))
        maskf = (dist <= _THR).astype(jnp.float32)
        sl = sl_ref[rows, :] * inv_t
        tl = gl_ref[rows, 0:_C] * inv_t
        sm = jnp.max(sl, axis=1, keepdims=True)
        s_lse = jnp.log(jnp.sum(jnp.exp(sl - sm), axis=1, keepdims=True)) + sm
        tm = jnp.max(tl, axis=1, keepdims=True)
        te = jnp.exp(tl - tm)
        tsum = jnp.sum(te, axis=1, keepdims=True)
        t_lse = jnp.log(tsum) + tm
        t_prob = te / tsum
        kl_per = jnp.sum(t_prob * ((tl - t_lse) - (sl - s_lse)),
                         axis=1, keepdims=True)
        kl_sum = kl_sum + jnp.sum(kl_per * maskf, keepdims=True)
        n_sum = n_sum + jnp.sum(maskf, keepdims=True)
    loss = jnp.where(n_sum > 0.0, kl_sum / jnp.maximum(n_sum, 1.0), 0.0)
    o_ref[...] = loss * (_TEMP * _TEMP * _KL_WEIGHT)


def _kl_loss(s_coord, s_logits, g_logits, g_coord):
    return pl.pallas_call(
        _kl_body,
        in_specs=[
            pl.BlockSpec((_NS, 3), lambda: (0, 0)),
            pl.BlockSpec((_NS, _C), lambda: (0, 0)),
            pl.BlockSpec((_NS, 128), lambda: (0, 0)),
            pl.BlockSpec((_NS, 128), lambda: (0, 0)),
        ],
        out_specs=pl.BlockSpec((1, 1), lambda: (0, 0)),
        out_shape=jax.ShapeDtypeStruct((1, 1), jnp.float32),
    )(s_coord, s_logits, g_logits, g_coord)


def kernel(s_coord, t_coord, s_logits, t_logits):
    col2d = _nn_cols(s_coord, t_coord.T)
    col = col2d.reshape(_NS)
    tl_pad = jnp.pad(t_logits, ((0, 0), (0, 128 - _C)))
    tc_pad = jnp.pad(t_coord, ((0, 0), (0, 125)))
    g_logits, g_coord = _make_sc_gather()(tl_pad, tc_pad, col)
    out = _kl_loss(s_coord, s_logits, g_logits, g_coord)
    return out[0, 0]


# speedup vs baseline: 1.3547x; 1.1174x over previous
import functools

import jax
import jax.numpy as jnp
from jax import lax
from jax.experimental import pallas as pl
from jax.experimental.pallas import tpu as pltpu
from jax.experimental.pallas import tpu_sc as plsc

_THR = 0.05
_TEMP = 2.0
_KL_WEIGHT = 0.2

_NS = 8192
_NT = 8192
_C = 22

_SBLK = 1024
_TCHUNK = 1024
_RBLK = 1024


def _nn_body(s_ref, tt_ref, oi_ref, od_ref):
    s3x2 = s_ref[...] * 2.0
    sx = s_ref[:, 0:1]
    sy = s_ref[:, 1:2]
    sz = s_ref[:, 2:3]
    s2 = sx * sx + sy * sy + sz * sz
    iota = lax.broadcasted_iota(jnp.int32, (_SBLK, _TCHUNK), 1)
    run_min = jnp.full((_SBLK, 1), jnp.inf, jnp.float32)
    run_idx = jnp.zeros((_SBLK, 1), jnp.int32)
    for c in range(_NT // _TCHUNK):
        cols = pl.ds(c * _TCHUNK, _TCHUNK)
        tx = tt_ref[0:1, cols]
        ty = tt_ref[1:2, cols]
        tz = tt_ref[2:3, cols]
        t2 = tx * tx + ty * ty + tz * tz
        dot2 = lax.dot_general(s3x2, tt_ref[:, cols],
                               (((1,), (0,)), ((), ())),
                               preferred_element_type=jnp.float32)
        d2 = s2 - dot2 + t2
        cmin = jnp.min(d2, axis=1, keepdims=True)
        cidx = jnp.min(jnp.where(d2 == cmin, iota, _NT), axis=1,
                       keepdims=True) + c * _TCHUNK
        better = cmin < run_min
        run_idx = jnp.where(better, cidx, run_idx)
        run_min = jnp.minimum(run_min, cmin)
    oi_ref[...] = run_idx
    od_ref[...] = run_min


def _nn_cols(s_coord, t_coord_t):
    return pl.pallas_call(
        _nn_body,
        grid=(_NS // _SBLK,),
        in_specs=[
            pl.BlockSpec((_SBLK, 3), lambda i: (i, 0)),
            pl.BlockSpec((3, _NT), lambda i: (0, 0)),
        ],
        out_specs=[pl.BlockSpec((_SBLK, 1), lambda i: (i, 0)),
                   pl.BlockSpec((_SBLK, 1), lambda i: (i, 0))],
        out_shape=[jax.ShapeDtypeStruct((_NS, 1), jnp.int32),
                   jax.ShapeDtypeStruct((_NS, 1), jnp.float32)],
        compiler_params=pltpu.CompilerParams(
            dimension_semantics=("parallel",)),
    )(s_coord, t_coord_t)


def _make_sc_gather():
    info = plsc.get_sparse_core_info()
    nw = info.num_cores * info.num_subcores
    b_per_w = _NS // nw
    mesh = plsc.VectorSubcoreMesh(core_axis_name="c", subcore_axis_name="s")

    @functools.partial(
        pl.kernel,
        out_type=jax.ShapeDtypeStruct((_NS, 128), jnp.float32),
        mesh=mesh,
        scratch_types=[pltpu.VMEM((b_per_w,), jnp.int32),
                       pltpu.VMEM((b_per_w, 128), jnp.float32),
                       pltpu.SemaphoreType.DMA],
    )
    def gather_kernel(tl_hbm, idx_hbm, gl_hbm, idx_v, rows_l, sem_l):
        wid = lax.axis_index("s") * info.num_cores + lax.axis_index("c")
        base = wid * b_per_w
        pltpu.sync_copy(idx_hbm.at[pl.ds(base, b_per_w)], idx_v)
        pltpu.async_copy(tl_hbm.at[idx_v], rows_l, sem_l).wait()
        pltpu.sync_copy(rows_l, gl_hbm.at[pl.ds(base, b_per_w)])

    return gather_kernel


def _kl_body(d2_ref, sl_ref, gl_ref, o_ref):
    kl_sum = jnp.zeros((1, 1), jnp.float32)
    n_sum = jnp.zeros((1, 1), jnp.float32)
    inv_t = 1.0 / _TEMP
    for c in range(_NS // _RBLK):
        rows = pl.ds(c * _RBLK, _RBLK)
        dist = jnp.sqrt(jnp.maximum(d2_ref[rows, :], 0.0))
        maskf = (dist <= _THR).astype(jnp.float32)
        sl = sl_ref[rows, :] * inv_t
        tl = gl_ref[rows, 0:_C] * inv_t
        sm = jnp.max(sl, axis=1, keepdims=True)
        s_lse = jnp.log(jnp.sum(jnp.exp(sl - sm), axis=1, keepdims=True)) + sm
        tm = jnp.max(tl, axis=1, keepdims=True)
        te = jnp.exp(tl - tm)
        tsum = jnp.sum(te, axis=1, keepdims=True)
        t_lse = jnp.log(tsum) + tm
        t_prob = te / tsum
        kl_per = jnp.sum(t_prob * ((tl - t_lse) - (sl - s_lse)),
                         axis=1, keepdims=True)
        kl_sum = kl_sum + jnp.sum(kl_per * maskf, keepdims=True)
        n_sum = n_sum + jnp.sum(maskf, keepdims=True)
    loss = jnp.where(n_sum > 0.0, kl_sum / jnp.maximum(n_sum, 1.0), 0.0)
    o_ref[...] = loss * (_TEMP * _TEMP * _KL_WEIGHT)


def _kl_loss(d2col, s_logits, g_logits):
    return pl.pallas_call(
        _kl_body,
        in_specs=[
            pl.BlockSpec((_NS, 1), lambda: (0, 0)),
            pl.BlockSpec((_NS, _C), lambda: (0, 0)),
            pl.BlockSpec((_NS, 128), lambda: (0, 0)),
        ],
        out_specs=pl.BlockSpec((1, 1), lambda: (0, 0)),
        out_shape=jax.ShapeDtypeStruct((1, 1), jnp.float32),
    )(d2col, s_logits, g_logits)


def kernel(s_coord, t_coord, s_logits, t_logits):
    col2d, d2col = _nn_cols(s_coord, t_coord.T)
    col = col2d.reshape(_NS)
    tl_pad = jnp.pad(t_logits, ((0, 0), (0, 128 - _C)))
    g_logits = _make_sc_gather()(tl_pad, col)
    out = _kl_loss(d2col, s_logits, g_logits)
    return out[0, 0]


# speedup vs baseline: 1.3548x; 1.0001x over previous
import functools

import jax
import jax.numpy as jnp
from jax import lax
from jax.experimental import pallas as pl
from jax.experimental.pallas import tpu as pltpu
from jax.experimental.pallas import tpu_sc as plsc

_THR = 0.05
_TEMP = 2.0
_KL_WEIGHT = 0.2

_NS = 8192
_NT = 8192
_C = 22

_SBLK = 1024
_TCHUNK = 1024
_RBLK = 1024


def _nn_body(s_ref, tt_ref, oi_ref, od_ref):
    s3x2 = s_ref[...] * 2.0
    sx = s_ref[:, 0:1]
    sy = s_ref[:, 1:2]
    sz = s_ref[:, 2:3]
    s2 = sx * sx + sy * sy + sz * sz
    iota = lax.broadcasted_iota(jnp.int32, (_SBLK, _TCHUNK), 1)
    run_min = jnp.full((_SBLK, 1), jnp.inf, jnp.float32)
    run_idx = jnp.zeros((_SBLK, 1), jnp.int32)
    for c in range(_NT // _TCHUNK):
        cols = pl.ds(c * _TCHUNK, _TCHUNK)
        tx = tt_ref[0:1, cols]
        ty = tt_ref[1:2, cols]
        tz = tt_ref[2:3, cols]
        t2 = tx * tx + ty * ty + tz * tz
        dot2 = lax.dot_general(s3x2, tt_ref[:, cols],
                               (((1,), (0,)), ((), ())),
                               preferred_element_type=jnp.float32)
        d2 = s2 - dot2 + t2
        cmin = jnp.min(d2, axis=1, keepdims=True)
        cidx = jnp.min(jnp.where(d2 == cmin, iota, _NT), axis=1,
                       keepdims=True) + c * _TCHUNK
        better = cmin < run_min
        run_idx = jnp.where(better, cidx, run_idx)
        run_min = jnp.minimum(run_min, cmin)
    oi_ref[...] = run_idx
    od_ref[...] = run_min


def _nn_cols(s_coord, t_coord_t):
    return pl.pallas_call(
        _nn_body,
        grid=(_NS // _SBLK,),
        in_specs=[
            pl.BlockSpec((_SBLK, 3), lambda i: (i, 0)),
            pl.BlockSpec((3, _NT), lambda i: (0, 0)),
        ],
        out_specs=[pl.BlockSpec((_SBLK, 1), lambda i: (i, 0)),
                   pl.BlockSpec((_SBLK, 1), lambda i: (i, 0))],
        out_shape=[jax.ShapeDtypeStruct((_NS, 1), jnp.int32),
                   jax.ShapeDtypeStruct((_NS, 1), jnp.float32)],
        compiler_params=pltpu.CompilerParams(
            dimension_semantics=("arbitrary",)),
    )(s_coord, t_coord_t)


def _make_sc_gather():
    info = plsc.get_sparse_core_info()
    nw = info.num_cores * info.num_subcores
    b_per_w = _NS // nw
    mesh = plsc.VectorSubcoreMesh(core_axis_name="c", subcore_axis_name="s")

    @functools.partial(
        pl.kernel,
        out_type=jax.ShapeDtypeStruct((_NS, 128), jnp.float32),
        mesh=mesh,
        scratch_types=[pltpu.VMEM((b_per_w,), jnp.int32),
                       pltpu.VMEM((b_per_w, 128), jnp.float32),
                       pltpu.SemaphoreType.DMA],
    )
    def gather_kernel(tl_hbm, idx_hbm, gl_hbm, idx_v, rows_l, sem_l):
        wid = lax.axis_index("s") * info.num_cores + lax.axis_index("c")
        base = wid * b_per_w
        pltpu.sync_copy(idx_hbm.at[pl.ds(base, b_per_w)], idx_v)
        pltpu.async_copy(tl_hbm.at[idx_v], rows_l, sem_l).wait()
        pltpu.sync_copy(rows_l, gl_hbm.at[pl.ds(base, b_per_w)])

    return gather_kernel


def _kl_body(d2_ref, sl_ref, gl_ref, o_ref):
    kl_sum = jnp.zeros((1, 1), jnp.float32)
    n_sum = jnp.zeros((1, 1), jnp.float32)
    inv_t = 1.0 / _TEMP
    for c in range(_NS // _RBLK):
        rows = pl.ds(c * _RBLK, _RBLK)
        dist = jnp.sqrt(jnp.maximum(d2_ref[rows, :], 0.0))
        maskf = (dist <= _THR).astype(jnp.float32)
        sl = sl_ref[rows, :] * inv_t
        tl = gl_ref[rows, 0:_C] * inv_t
        sm = jnp.max(sl, axis=1, keepdims=True)
        s_lse = jnp.log(jnp.sum(jnp.exp(sl - sm), axis=1, keepdims=True)) + sm
        tm = jnp.max(tl, axis=1, keepdims=True)
        te = jnp.exp(tl - tm)
        tsum = jnp.sum(te, axis=1, keepdims=True)
        t_lse = jnp.log(tsum) + tm
        t_prob = te / tsum
        kl_per = jnp.sum(t_prob * ((tl - t_lse) - (sl - s_lse)),
                         axis=1, keepdims=True)
        kl_sum = kl_sum + jnp.sum(kl_per * maskf, keepdims=True)
        n_sum = n_sum + jnp.sum(maskf, keepdims=True)
    loss = jnp.where(n_sum > 0.0, kl_sum / jnp.maximum(n_sum, 1.0), 0.0)
    o_ref[...] = loss * (_TEMP * _TEMP * _KL_WEIGHT)


def _kl_loss(d2col, s_logits, g_logits):
    return pl.pallas_call(
        _kl_body,
        in_specs=[
            pl.BlockSpec((_NS, 1), lambda: (0, 0)),
            pl.BlockSpec((_NS, _C), lambda: (0, 0)),
            pl.BlockSpec((_NS, 128), lambda: (0, 0)),
        ],
        out_specs=pl.BlockSpec((1, 1), lambda: (0, 0)),
        out_shape=jax.ShapeDtypeStruct((1, 1), jnp.float32),
    )(d2col, s_logits, g_logits)


def kernel(s_coord, t_coord, s_logits, t_logits):
    col2d, d2col = _nn_cols(s_coord, t_coord.T)
    col = col2d.reshape(_NS)
    tl_pad = jnp.pad(t_logits, ((0, 0), (0, 128 - _C)))
    g_logits = _make_sc_gather()(tl_pad, col)
    out = _kl_loss(d2col, s_logits, g_logits)
    return out[0, 0]


# speedup vs baseline: 1.5472x; 1.1420x over previous
import functools

import jax
import jax.numpy as jnp
from jax import lax
from jax.experimental import pallas as pl
from jax.experimental.pallas import tpu as pltpu
from jax.experimental.pallas import tpu_sc as plsc

_THR = 0.05
_TEMP = 2.0
_KL_WEIGHT = 0.2

_NS = 8192
_NT = 8192
_C = 22

_SBLK = 1024
_TCHUNK = 1024
_RBLK = 1024


def _nn_body(s_ref, tt_ref, oi_ref, od_ref):
    s3x2 = s_ref[...] * 2.0
    sx = s_ref[:, 0:1]
    sy = s_ref[:, 1:2]
    sz = s_ref[:, 2:3]
    s2 = sx * sx + sy * sy + sz * sz
    iota = lax.broadcasted_iota(jnp.int32, (_SBLK, _TCHUNK), 1).astype(
        jnp.float32)
    run_min = jnp.full((_SBLK, 1), jnp.inf, jnp.float32)
    run_idx = jnp.zeros((_SBLK, 1), jnp.float32)
    for c in range(_NT // _TCHUNK):
        cols = pl.ds(c * _TCHUNK, _TCHUNK)
        tx = tt_ref[0:1, cols]
        ty = tt_ref[1:2, cols]
        tz = tt_ref[2:3, cols]
        t2 = tx * tx + ty * ty + tz * tz
        dot2 = lax.dot_general(s3x2, tt_ref[:, cols],
                               (((1,), (0,)), ((), ())),
                               preferred_element_type=jnp.float32)
        d2 = s2 - dot2 + t2
        cmin = jnp.min(d2, axis=1, keepdims=True)
        cidx = jnp.min(jnp.where(d2 == cmin, iota, float(_NT)), axis=1,
                       keepdims=True) + float(_TCHUNK) * c
        better = cmin < run_min
        run_idx = jnp.where(better, cidx, run_idx)
        run_min = jnp.minimum(run_min, cmin)
    oi_ref[...] = run_idx.astype(jnp.int32)
    od_ref[...] = run_min


def _nn_cols(s_coord, t_coord_t):
    return pl.pallas_call(
        _nn_body,
        grid=(_NS // _SBLK,),
        in_specs=[
            pl.BlockSpec((_SBLK, 3), lambda i: (i, 0)),
            pl.BlockSpec((3, _NT), lambda i: (0, 0)),
        ],
        out_specs=[pl.BlockSpec((_SBLK, 1), lambda i: (i, 0)),
                   pl.BlockSpec((_SBLK, 1), lambda i: (i, 0))],
        out_shape=[jax.ShapeDtypeStruct((_NS, 1), jnp.int32),
                   jax.ShapeDtypeStruct((_NS, 1), jnp.float32)],
        compiler_params=pltpu.CompilerParams(
            dimension_semantics=("arbitrary",)),
    )(s_coord, t_coord_t)


def _make_sc_gather():
    info = plsc.get_sparse_core_info()
    nw = info.num_cores * info.num_subcores
    b_per_w = _NS // nw
    mesh = plsc.VectorSubcoreMesh(core_axis_name="c", subcore_axis_name="s")

    @functools.partial(
        pl.kernel,
        out_type=jax.ShapeDtypeStruct((_NS, 128), jnp.float32),
        mesh=mesh,
        scratch_types=[pltpu.VMEM((b_per_w,), jnp.int32),
                       pltpu.VMEM((b_per_w, 128), jnp.float32),
                       pltpu.SemaphoreType.DMA],
    )
    def gather_kernel(tl_hbm, idx_hbm, gl_hbm, idx_v, rows_l, sem_l):
        wid = lax.axis_index("s") * info.num_cores + lax.axis_index("c")
        base = wid * b_per_w
        pltpu.sync_copy(idx_hbm.at[pl.ds(base, b_per_w)], idx_v)
        pltpu.async_copy(tl_hbm.at[idx_v], rows_l, sem_l).wait()
        pltpu.sync_copy(rows_l, gl_hbm.at[pl.ds(base, b_per_w)])

    return gather_kernel


def _kl_body(d2_ref, sl_ref, gl_ref, o_ref):
    kl_sum = jnp.zeros((1, 1), jnp.float32)
    n_sum = jnp.zeros((1, 1), jnp.float32)
    inv_t = 1.0 / _TEMP
    for c in range(_NS // _RBLK):
        rows = pl.ds(c * _RBLK, _RBLK)
        dist = jnp.sqrt(jnp.maximum(d2_ref[rows, :], 0.0))
        maskf = (dist <= _THR).astype(jnp.float32)
        sl = sl_ref[rows, :] * inv_t
        tl = gl_ref[rows, 0:_C] * inv_t
        sm = jnp.max(sl, axis=1, keepdims=True)
        s_lse = jnp.log(jnp.sum(jnp.exp(sl - sm), axis=1, keepdims=True)) + sm
        tm = jnp.max(tl, axis=1, keepdims=True)
        te = jnp.exp(tl - tm)
        tsum = jnp.sum(te, axis=1, keepdims=True)
        t_lse = jnp.log(tsum) + tm
        t_prob = te / tsum
        kl_per = jnp.sum(t_prob * ((tl - t_lse) - (sl - s_lse)),
                         axis=1, keepdims=True)
        kl_sum = kl_sum + jnp.sum(kl_per * maskf, keepdims=True)
        n_sum = n_sum + jnp.sum(maskf, keepdims=True)
    loss = jnp.where(n_sum > 0.0, kl_sum / jnp.maximum(n_sum, 1.0), 0.0)
    o_ref[...] = loss * (_TEMP * _TEMP * _KL_WEIGHT)


def _kl_loss(d2col, s_logits, g_logits):
    return pl.pallas_call(
        _kl_body,
        in_specs=[
            pl.BlockSpec((_NS, 1), lambda: (0, 0)),
            pl.BlockSpec((_NS, _C), lambda: (0, 0)),
            pl.BlockSpec((_NS, 128), lambda: (0, 0)),
        ],
        out_specs=pl.BlockSpec((1, 1), lambda: (0, 0)),
        out_shape=jax.ShapeDtypeStruct((1, 1), jnp.float32),
    )(d2col, s_logits, g_logits)


def kernel(s_coord, t_coord, s_logits, t_logits):
    col2d, d2col = _nn_cols(s_coord, t_coord.T)
    col = col2d.reshape(_NS)
    tl_pad = jnp.pad(t_logits, ((0, 0), (0, 128 - _C)))
    g_logits = _make_sc_gather()(tl_pad, col)
    out = _kl_loss(d2col, s_logits, g_logits)
    return out[0, 0]
